# baseline (device time: 82946 ns/iter reference)
import jax
import jax.numpy as jnp
from jax import lax
from jax.experimental import pallas as pl
from jax.experimental.pallas import tpu as pltpu

N_DEV = 4
E_LOCAL = 4
E_TOTAL = N_DEV * E_LOCAL


def kernel(x, router_W, route_idx, expert_W):
    n_tok, d_model = x.shape
    e_loc, _, d_ff = expert_W.shape

    def body(x_ref, rw_ref, idx_ref, ew_ref, out_ref, comm_ref, send_sems, recv_sems):
        my = lax.axis_index("i")
        right = lax.rem(my + 1, N_DEV)
        left = lax.rem(my + N_DEV - 1, N_DEV)

        barrier_sem = pltpu.get_barrier_semaphore()
        for nbr in (left, right):
            pl.semaphore_signal(
                barrier_sem, inc=1,
                device_id=(nbr,), device_id_type=pl.DeviceIdType.MESH,
            )
        pl.semaphore_wait(barrier_sem, 2)

        rdma0 = pltpu.make_async_remote_copy(
            src_ref=ew_ref,
            dst_ref=comm_ref.at[1],
            send_sem=send_sems.at[0],
            recv_sem=recv_sems.at[0],
            device_id=(right,),
            device_id_type=pl.DeviceIdType.MESH,
        )
        rdma0.start()

        xv = x_ref[:, :]
        scores = jnp.dot(xv, rw_ref[:, :], preferred_element_type=jnp.float32)
        probs = jnp.exp(scores - jnp.max(scores, axis=-1, keepdims=True))
        e_ids = lax.broadcasted_iota(jnp.int32, (n_tok, E_TOTAL), 1)
        sel = (e_ids == idx_ref[:, 0:1]) | (e_ids == idx_ref[:, 1:2])
        gsel = jnp.where(sel, probs, 0.0)
        gates = gsel / jnp.sum(gsel, axis=-1, keepdims=True)

        row = lax.broadcasted_iota(jnp.int32, (E_TOTAL, E_TOTAL), 0)
        col = lax.broadcasted_iota(jnp.int32, (E_TOTAL, E_TOTAL), 1)
        owner = lax.rem(my - col // E_LOCAL + N_DEV, N_DEV)
        perm = (row == owner * E_LOCAL + lax.rem(col, E_LOCAL)).astype(jnp.float32)
        gates_h = jnp.dot(gates, perm, preferred_element_type=jnp.float32)

        rdmas = [rdma0]
        acc = jnp.zeros((n_tok, d_ff), jnp.float32)
        for h in range(N_DEV):
            for j in range(E_LOCAL):
                g = gates_h[:, h * E_LOCAL + j : h * E_LOCAL + j + 1]
                wj = ew_ref[j, :, :] if h == 0 else comm_ref[h, j, :, :]
                acc = acc + jnp.dot(xv * g, wj, preferred_element_type=jnp.float32)
            if h < N_DEV - 1:
                rdmas[h].wait_recv()
                if h < N_DEV - 2:
                    nxt = pltpu.make_async_remote_copy(
                        src_ref=comm_ref.at[h + 1],
                        dst_ref=comm_ref.at[h + 2],
                        send_sem=send_sems.at[h + 1],
                        recv_sem=recv_sems.at[h + 1],
                        device_id=(right,),
                        device_id_type=pl.DeviceIdType.MESH,
                    )
                    nxt.start()
                    rdmas.append(nxt)

        out_ref[:, :] = acc

        for r in rdmas:
            r.wait_send()

    return pl.pallas_call(
        body,
        out_shape=jax.ShapeDtypeStruct((n_tok, d_ff), jnp.float32),
        in_specs=[
            pl.BlockSpec(memory_space=pltpu.VMEM),
            pl.BlockSpec(memory_space=pltpu.VMEM),
            pl.BlockSpec(memory_space=pltpu.VMEM),
            pl.BlockSpec(memory_space=pltpu.VMEM),
        ],
        out_specs=pl.BlockSpec(memory_space=pltpu.VMEM),
        scratch_shapes=[
            pltpu.VMEM((N_DEV, e_loc, d_model, d_ff), jnp.float32),
            pltpu.SemaphoreType.DMA((N_DEV - 1,)),
            pltpu.SemaphoreType.DMA((N_DEV - 1,)),
        ],
        compiler_params=pltpu.CompilerParams(collective_id=0),
    )(x, router_W, route_idx, expert_W)


# device time: 78986 ns/iter; 1.0501x vs baseline; 1.0501x over previous
import jax
import jax.numpy as jnp
from jax import lax
from jax.experimental import pallas as pl
from jax.experimental.pallas import tpu as pltpu

N_DEV = 4
E_LOCAL = 4
E_TOTAL = N_DEV * E_LOCAL


def kernel(x, router_W, route_idx, expert_W):
    n_tok, d_model = x.shape
    e_loc, _, d_ff = expert_W.shape

    def body(x_ref, rw_ref, idx_ref, ew_ref, out_ref, comm_ref, send_sems, recv_sems):
        my = lax.axis_index("i")
        right = lax.rem(my + 1, N_DEV)
        left = lax.rem(my + N_DEV - 1, N_DEV)

        barrier_sem = pltpu.get_barrier_semaphore()
        for nbr in (left, right):
            pl.semaphore_signal(
                barrier_sem, inc=1,
                device_id=(nbr,), device_id_type=pl.DeviceIdType.MESH,
            )
        pl.semaphore_wait(barrier_sem, 2)

        def make_rdma(h, j):
            return pltpu.make_async_remote_copy(
                src_ref=ew_ref.at[j] if h == 0 else comm_ref.at[h, j],
                dst_ref=comm_ref.at[h + 1, j],
                send_sem=send_sems.at[h, j],
                recv_sem=recv_sems.at[h, j],
                device_id=(right,),
                device_id_type=pl.DeviceIdType.MESH,
            )

        rdmas = [[make_rdma(h, j) for j in range(E_LOCAL)]
                 for h in range(N_DEV - 1)]

        for j in range(E_LOCAL):
            rdmas[0][j].start()

        xv = x_ref[:, :]
        scores = jnp.dot(xv, rw_ref[:, :], preferred_element_type=jnp.float32)
        probs = jnp.exp(scores - jnp.max(scores, axis=-1, keepdims=True))
        e_ids = lax.broadcasted_iota(jnp.int32, (n_tok, E_TOTAL), 1)
        sel = (e_ids == idx_ref[:, 0:1]) | (e_ids == idx_ref[:, 1:2])
        gsel = jnp.where(sel, probs, 0.0)
        gates = gsel / jnp.sum(gsel, axis=-1, keepdims=True)

        row = lax.broadcasted_iota(jnp.int32, (E_TOTAL, E_TOTAL), 0)
        col = lax.broadcasted_iota(jnp.int32, (E_TOTAL, E_TOTAL), 1)
        owner = lax.rem(my - col // E_LOCAL + N_DEV, N_DEV)
        perm = (row == owner * E_LOCAL + lax.rem(col, E_LOCAL)).astype(jnp.float32)
        gates_h = jnp.dot(gates, perm, preferred_element_type=jnp.float32)

        acc = jnp.zeros((n_tok, d_ff), jnp.float32)
        for h in range(N_DEV):
            for j in range(E_LOCAL):
                if h > 0:
                    rdmas[h - 1][j].wait_recv()
                    if h < N_DEV - 1:
                        rdmas[h][j].start()
                g = gates_h[:, h * E_LOCAL + j : h * E_LOCAL + j + 1]
                wj = ew_ref[j, :, :] if h == 0 else comm_ref[h, j, :, :]
                acc = acc + jnp.dot(xv * g, wj, preferred_element_type=jnp.float32)

        out_ref[:, :] = acc

        for hop in rdmas:
            for r in hop:
                r.wait_send()

    return pl.pallas_call(
        body,
        out_shape=jax.ShapeDtypeStruct((n_tok, d_ff), jnp.float32),
        in_specs=[
            pl.BlockSpec(memory_space=pltpu.VMEM),
            pl.BlockSpec(memory_space=pltpu.VMEM),
            pl.BlockSpec(memory_space=pltpu.VMEM),
            pl.BlockSpec(memory_space=pltpu.VMEM),
        ],
        out_specs=pl.BlockSpec(memory_space=pltpu.VMEM),
        scratch_shapes=[
            pltpu.VMEM((N_DEV, e_loc, d_model, d_ff), jnp.float32),
            pltpu.SemaphoreType.DMA((N_DEV - 1, E_LOCAL)),
            pltpu.SemaphoreType.DMA((N_DEV - 1, E_LOCAL)),
        ],
        compiler_params=pltpu.CompilerParams(collective_id=0),
    )(x, router_W, route_idx, expert_W)


# device time: 45262 ns/iter; 1.8326x vs baseline; 1.7451x over previous
import jax
import jax.numpy as jnp
from jax import lax
from jax.experimental import pallas as pl
from jax.experimental.pallas import tpu as pltpu

N_DEV = 4
E_LOCAL = 4
E_TOTAL = N_DEV * E_LOCAL


def kernel(x, router_W, route_idx, expert_W):
    n_tok, d_model = x.shape
    e_loc, _, d_ff = expert_W.shape

    def body(x_ref, rw_ref, idx_ref, ew_ref, out_ref, comm_ref, send_sems, recv_sems):
        my = lax.axis_index("i")
        right = lax.rem(my + 1, N_DEV)
        left = lax.rem(my + N_DEV - 1, N_DEV)

        barrier_sem = pltpu.get_barrier_semaphore()
        for nbr in (left, right):
            pl.semaphore_signal(
                barrier_sem, inc=1,
                device_id=(nbr,), device_id_type=pl.DeviceIdType.MESH,
            )
        pl.semaphore_wait(barrier_sem, 2)

        def make_rdma(h, j):
            return pltpu.make_async_remote_copy(
                src_ref=comm_ref.at[h, j],
                dst_ref=comm_ref.at[h + 1, j],
                send_sem=send_sems.at[h, j],
                recv_sem=recv_sems.at[h, j],
                device_id=(right,),
                device_id_type=pl.DeviceIdType.MESH,
            )

        rdmas = [[make_rdma(h, j) for j in range(E_LOCAL)]
                 for h in range(N_DEV - 1)]

        for j in range(E_LOCAL):
            comm_ref[0, j, :, :] = ew_ref[j, :, :].astype(jnp.bfloat16)
            rdmas[0][j].start()

        xv = x_ref[:, :]
        scores = jnp.dot(xv, rw_ref[:, :], preferred_element_type=jnp.float32)
        probs = jnp.exp(scores - jnp.max(scores, axis=-1, keepdims=True))
        e_ids = lax.broadcasted_iota(jnp.int32, (n_tok, E_TOTAL), 1)
        sel = (e_ids == idx_ref[:, 0:1]) | (e_ids == idx_ref[:, 1:2])
        gsel = jnp.where(sel, probs, 0.0)
        gates = gsel / jnp.sum(gsel, axis=-1, keepdims=True)

        row = lax.broadcasted_iota(jnp.int32, (E_TOTAL, E_TOTAL), 0)
        col = lax.broadcasted_iota(jnp.int32, (E_TOTAL, E_TOTAL), 1)
        owner = lax.rem(my - col // E_LOCAL + N_DEV, N_DEV)
        perm = (row == owner * E_LOCAL + lax.rem(col, E_LOCAL)).astype(jnp.float32)
        gates_h = jnp.dot(gates, perm, preferred_element_type=jnp.float32)

        xv_bf = xv.astype(jnp.bfloat16)
        acc = jnp.zeros((n_tok, d_ff), jnp.float32)
        for h in range(N_DEV):
            for j in range(E_LOCAL):
                if h > 0:
                    rdmas[h - 1][j].wait_recv()
                    if h < N_DEV - 1:
                        rdmas[h][j].start()
                g = gates_h[:, h * E_LOCAL + j : h * E_LOCAL + j + 1]
                z = jnp.dot(xv_bf, comm_ref[h, j, :, :],
                            preferred_element_type=jnp.float32)
                acc = acc + g * z

        out_ref[:, :] = acc

        for hop in rdmas:
            for r in hop:
                r.wait_send()

    return pl.pallas_call(
        body,
        out_shape=jax.ShapeDtypeStruct((n_tok, d_ff), jnp.float32),
        in_specs=[
            pl.BlockSpec(memory_space=pltpu.VMEM),
            pl.BlockSpec(memory_space=pltpu.VMEM),
            pl.BlockSpec(memory_space=pltpu.VMEM),
            pl.BlockSpec(memory_space=pltpu.VMEM),
        ],
        out_specs=pl.BlockSpec(memory_space=pltpu.VMEM),
        scratch_shapes=[
            pltpu.VMEM((N_DEV, e_loc, d_model, d_ff), jnp.bfloat16),
            pltpu.SemaphoreType.DMA((N_DEV - 1, E_LOCAL)),
            pltpu.SemaphoreType.DMA((N_DEV - 1, E_LOCAL)),
        ],
        compiler_params=pltpu.CompilerParams(collective_id=0),
    )(x, router_W, route_idx, expert_W)


# device time: 28424 ns/iter; 2.9182x vs baseline; 1.5924x over previous
import jax
import jax.numpy as jnp
from jax import lax
from jax.experimental import pallas as pl
from jax.experimental.pallas import tpu as pltpu

N_DEV = 4
E_LOCAL = 4
E_TOTAL = N_DEV * E_LOCAL


def kernel(x, router_W, route_idx, expert_W):
    n_tok, d_model = x.shape
    e_loc, _, d_ff = expert_W.shape

    def body(x_ref, rw_ref, idx_ref, ew_ref, out_ref, comm_ref, send_sems, recv_sems):
        my = lax.axis_index("i")
        right = lax.rem(my + 1, N_DEV)
        left = lax.rem(my + N_DEV - 1, N_DEV)

        barrier_sem = pltpu.get_barrier_semaphore()
        for nbr in (left, right):
            pl.semaphore_signal(
                barrier_sem, inc=1,
                device_id=(nbr,), device_id_type=pl.DeviceIdType.MESH,
            )
        pl.semaphore_wait(barrier_sem, 2)

        def make_rdma(h, j):
            return pltpu.make_async_remote_copy(
                src_ref=comm_ref.at[h, j],
                dst_ref=comm_ref.at[h + 1, j],
                send_sem=send_sems.at[h, j],
                recv_sem=recv_sems.at[h, j],
                device_id=(right,) if j < 2 else (left,),
                device_id_type=pl.DeviceIdType.MESH,
            )

        rdmas = [[make_rdma(h, j) for j in range(E_LOCAL)]
                 for h in range(N_DEV - 1)]

        for j in range(E_LOCAL):
            comm_ref[0, j, :, :] = ew_ref[j, :, :].astype(jnp.bfloat16)
            rdmas[0][j].start()

        xv = x_ref[:, :]
        scores = jnp.dot(xv, rw_ref[:, :], preferred_element_type=jnp.float32)
        probs = jnp.exp(scores - jnp.max(scores, axis=-1, keepdims=True))
        e_ids = lax.broadcasted_iota(jnp.int32, (n_tok, E_TOTAL), 1)
        sel = (e_ids == idx_ref[:, 0:1]) | (e_ids == idx_ref[:, 1:2])
        gsel = jnp.where(sel, probs, 0.0)
        gates = gsel / jnp.sum(gsel, axis=-1, keepdims=True)

        row = lax.broadcasted_iota(jnp.int32, (E_TOTAL, E_TOTAL), 0)
        col = lax.broadcasted_iota(jnp.int32, (E_TOTAL, E_TOTAL), 1)
        hh = col // E_LOCAL
        jj = lax.rem(col, E_LOCAL)
        owner = jnp.where(
            jj < 2,
            lax.rem(my - hh + N_DEV, N_DEV),
            lax.rem(my + hh, N_DEV),
        )
        perm = (row == owner * E_LOCAL + jj).astype(jnp.float32)
        gates_h = jnp.dot(gates, perm, preferred_element_type=jnp.float32)

        xv_bf = xv.astype(jnp.bfloat16)
        acc = jnp.zeros((n_tok, d_ff), jnp.float32)
        for h in range(N_DEV):
            for j in range(E_LOCAL):
                if h > 0:
                    rdmas[h - 1][j].wait_recv()
                    if h < N_DEV - 1:
                        rdmas[h][j].start()
                g = gates_h[:, h * E_LOCAL + j : h * E_LOCAL + j + 1]
                z = jnp.dot(xv_bf, comm_ref[h, j, :, :],
                            preferred_element_type=jnp.float32)
                acc = acc + g * z

        out_ref[:, :] = acc

        for hop in rdmas:
            for r in hop:
                r.wait_send()

    return pl.pallas_call(
        body,
        out_shape=jax.ShapeDtypeStruct((n_tok, d_ff), jnp.float32),
        in_specs=[
            pl.BlockSpec(memory_space=pltpu.VMEM),
            pl.BlockSpec(memory_space=pltpu.VMEM),
            pl.BlockSpec(memory_space=pltpu.VMEM),
            pl.BlockSpec(memory_space=pltpu.VMEM),
        ],
        out_specs=pl.BlockSpec(memory_space=pltpu.VMEM),
        scratch_shapes=[
            pltpu.VMEM((N_DEV, e_loc, d_model, d_ff), jnp.bfloat16),
            pltpu.SemaphoreType.DMA((N_DEV - 1, E_LOCAL)),
            pltpu.SemaphoreType.DMA((N_DEV - 1, E_LOCAL)),
        ],
        compiler_params=pltpu.CompilerParams(collective_id=0),
    )(x, router_W, route_idx, expert_W)
